# Initial kernel scaffold; baseline (speedup 1.0000x reference)
import jax
import jax.numpy as jnp
from jax.experimental import pallas as pl


def kernel(tokens, mask, embed_table, W, b):
    raise NotImplementedError("write your pallas kernel here")


# speedup vs baseline: 5.7327x; 5.7327x over previous
import functools

import jax
import jax.numpy as jnp
from jax import lax
from jax.experimental import pallas as pl
from jax.experimental.pallas import tpu as pltpu
from jax.experimental.pallas import tpu_sc as plsc

DIM = 32
OUT_DIM = 64
B = 4096
T = 50
NTOK = B * T

NC, NS = 2, 16
NW = NC * NS
TOK_PER_W = NTOK // NW
GATHER = 128
NG = TOK_PER_W // GATHER
K = 10
NSTEP = NG // K
CHUNK = K * GATHER

_sc_mesh = plsc.VectorSubcoreMesh(core_axis_name="c", subcore_axis_name="s")


@functools.partial(
    pl.kernel,
    out_type=jax.ShapeDtypeStruct((NTOK, DIM), jnp.float32),
    mesh=_sc_mesh,
    scratch_types=[
        pltpu.VMEM((NG, GATHER), jnp.int32),
        pltpu.VMEM((CHUNK, DIM), jnp.float32),
        pltpu.SemaphoreType.DMA,
    ],
    compiler_params=pltpu.CompilerParams(use_tc_tiling_on_sc=False),
)
def _sc_gather(tok_hbm, table_hbm, out_hbm, idx_v, rows_v, gsem):
    wid = lax.axis_index("s") * NC + lax.axis_index("c")
    pltpu.sync_copy(tok_hbm.at[wid], idx_v)
    base = wid * TOK_PER_W
    for s in range(NSTEP):
        copies = []
        for j in range(K):
            c = pltpu.async_copy(
                table_hbm.at[idx_v.at[s * K + j]],
                rows_v.at[pl.ds(j * GATHER, GATHER)],
                gsem,
            )
            copies.append(c)
        for c in copies:
            c.wait()
        pltpu.sync_copy(rows_v, out_hbm.at[pl.ds(base + s * CHUNK, CHUNK)])


BLK = 2048
NBLK = NTOK // BLK


def _proj_body(g_ref, m_ref, w_ref, b_ref, out_ref):
    g = g_ref[...]
    acc = lax.dot_general(
        g, w_ref[...],
        dimension_numbers=(((1,), (1,)), ((), ())),
        preferred_element_type=jnp.float32,
    )
    out_ref[...] = (acc + b_ref[...]) * m_ref[...]


_proj = pl.pallas_call(
    _proj_body,
    grid=(NBLK,),
    in_specs=[
        pl.BlockSpec((BLK, DIM), lambda i: (i, 0)),
        pl.BlockSpec((BLK, 1), lambda i: (i, 0)),
        pl.BlockSpec((OUT_DIM, DIM), lambda i: (0, 0)),
        pl.BlockSpec((1, OUT_DIM), lambda i: (0, 0)),
    ],
    out_specs=pl.BlockSpec((BLK, OUT_DIM), lambda i: (i, 0)),
    out_shape=jax.ShapeDtypeStruct((NTOK, OUT_DIM), jnp.float32),
)


def kernel(tokens, mask, embed_table, W, b):
    tok3d = tokens.reshape(NW, NG, GATHER).astype(jnp.int32)
    gathered = _sc_gather(tok3d, embed_table)
    maskf = mask.reshape(NTOK, 1).astype(jnp.float32)
    out = _proj(gathered, maskf, W, b.reshape(1, OUT_DIM))
    return out.reshape(B, T, OUT_DIM), mask


# speedup vs baseline: 7.9713x; 1.3905x over previous
import functools

import jax
import jax.numpy as jnp
from jax import lax
from jax.experimental import pallas as pl
from jax.experimental.pallas import tpu as pltpu
from jax.experimental.pallas import tpu_sc as plsc

DIM = 32
OUT_DIM = 64
B = 4096
T = 50
NQUAD = 250000

NC, NS = 2, 16
NW = NC * NS
NBLK = B // NW
L = 16

_sc_mesh = plsc.VectorSubcoreMesh(core_axis_name="c", subcore_axis_name="s")


@functools.partial(
    pl.kernel,
    out_type=jax.ShapeDtypeStruct((T, B, DIM), jnp.float32),
    mesh=_sc_mesh,
    scratch_types=[
        pltpu.VMEM((T, NBLK), jnp.int32),
        pltpu.VMEM((T, NBLK), jnp.int32),
        pltpu.VMEM((2, NBLK, 128), jnp.float32),
        pltpu.VMEM((2, NBLK, DIM), jnp.float32),
        pltpu.SemaphoreType.DMA,
        pltpu.SemaphoreType.DMA,
        pltpu.SemaphoreType.DMA,
        pltpu.SemaphoreType.DMA,
    ],
    compiler_params=pltpu.CompilerParams(use_tc_tiling_on_sc=True),
)
def _sc_gather(tokT_hbm, table_hbm, inter_hbm, tok_v, q_v, buf_v,
               slab_v, g0, g1, w0, w1):
    wid = lax.axis_index("s") * NC + lax.axis_index("c")
    n0 = wid * NBLK

    pltpu.sync_copy(tokT_hbm.at[:, pl.ds(n0, NBLK)], tok_v)

    def _pre(t, _):
        for g in range(NBLK // L):
            tok = tok_v[t, pl.ds(g * L, L)]
            q_v[t, pl.ds(g * L, L)] = lax.shift_right_logical(tok, 2)
        return 0

    lax.fori_loop(0, T, _pre, 0, unroll=False)

    gsems = (g0, g1)
    wsems = (w0, w1)

    def _gather(i, par):
        return pltpu.make_async_copy(
            table_hbm.at[q_v.at[i]], buf_v.at[par], gsems[par])

    def _slab_copy(i, par):
        return pltpu.make_async_copy(
            slab_v.at[par], inter_hbm.at[i, pl.ds(n0, NBLK), :], wsems[par])

    def _step(i, par):
        @pl.when(i + 1 < T)
        def _():
            _gather(i + 1, 1 - par).start()

        _gather(i, par).wait()

        @pl.when(i >= 2)
        def _():
            _slab_copy(i, par).wait()

        buf = buf_v.at[par]
        slab = slab_v.at[par]
        for g in range(NBLK // L):
            rsv = lax.bitwise_and(tok_v[i, pl.ds(g * L, L)], 3) * DIM
            for l in range(L):
                rs = rsv[l]
                j = g * L + l
                slab[j, pl.ds(0, L)] = buf[j, pl.ds(rs, L)]
                slab[j, pl.ds(L, L)] = buf[j, pl.ds(rs + L, L)]
        _slab_copy(i, par).start()
        return 0

    def _pair(k, _):
        _step(2 * k, 0)
        _step(2 * k + 1, 1)
        return 0

    _gather(0, 0).start()
    lax.fori_loop(0, T // 2, _pair, 0, unroll=False)

    _slab_copy(T - 2, 0).wait()
    _slab_copy(T - 1, 1).wait()


def _proj_body(g_ref, m_ref, w_ref, b_ref, out_ref):
    g = g_ref[0]
    acc = lax.dot_general(
        w_ref[...], g,
        dimension_numbers=(((1,), (1,)), ((), ())),
        preferred_element_type=jnp.float32,
    )
    out_ref[0] = (acc + b_ref[...]) * m_ref[0]


_proj = pl.pallas_call(
    _proj_body,
    grid=(T,),
    in_specs=[
        pl.BlockSpec((1, B, DIM), lambda t: (t, 0, 0)),
        pl.BlockSpec((1, 1, B), lambda t: (t, 0, 0)),
        pl.BlockSpec((OUT_DIM, DIM), lambda t: (0, 0)),
        pl.BlockSpec((OUT_DIM, 1), lambda t: (0, 0)),
    ],
    out_specs=pl.BlockSpec((1, OUT_DIM, B), lambda t: (t, 0, 0)),
    out_shape=jax.ShapeDtypeStruct((T, OUT_DIM, B), jnp.float32),
)


def kernel(tokens, mask, embed_table, W, b):
    tokT = tokens.T.astype(jnp.int32)
    table4 = embed_table.reshape(NQUAD, 128)
    inter = _sc_gather(tokT, table4)
    maskT = mask.T.astype(jnp.float32).reshape(T, 1, B)
    out = _proj(inter, maskT, W, b.reshape(OUT_DIM, 1))
    return jnp.transpose(out, (2, 0, 1)), mask


# speedup vs baseline: 12.4324x; 1.5596x over previous
import functools

import jax
import jax.numpy as jnp
from jax import lax
from jax.experimental import pallas as pl
from jax.experimental.pallas import tpu as pltpu
from jax.experimental.pallas import tpu_sc as plsc

DIM = 32
OUT_DIM = 64
B = 4096
T = 50
NROW = 1000000
NQUAD = 262144
OFF3 = 741376
K3_MIN = 786432

NC, NS = 2, 16
NW = NC * NS
NBLK = B // NW
L = 16

_sc_mesh = plsc.VectorSubcoreMesh(core_axis_name="c", subcore_axis_name="s")


@functools.partial(
    pl.kernel,
    out_type=jax.ShapeDtypeStruct((T, B, DIM), jnp.float32),
    mesh=_sc_mesh,
    scratch_types=[
        pltpu.VMEM((T, NBLK), jnp.int32),
        pltpu.VMEM((T, NBLK), jnp.int32),
        pltpu.VMEM((T, NBLK), jnp.int32),
        pltpu.VMEM((2, NBLK, 128), jnp.float32),
        pltpu.VMEM((2, NBLK, DIM), jnp.float32),
        pltpu.SemaphoreType.DMA,
        pltpu.SemaphoreType.DMA,
        pltpu.SemaphoreType.DMA,
        pltpu.SemaphoreType.DMA,
    ],
    compiler_params=pltpu.CompilerParams(use_tc_tiling_on_sc=True),
)
def _sc_gather(tokT_hbm, table_hbm, inter_hbm, tok_v, q_v, r_v, buf_v,
               slab_v, g0, g1, w0, w1):
    wid = lax.axis_index("s") * NC + lax.axis_index("c")
    n0 = wid * NBLK

    pltpu.sync_copy(tokT_hbm.at[:, pl.ds(n0, NBLK)], tok_v)

    def _pre(t, _):
        for g in range(NBLK // L):
            sl = pl.ds(g * L, L)
            tok = tok_v[t, sl]
            hi = tok >= K3_MIN
            k = jnp.where(hi, 3, lax.shift_right_logical(tok, 18))
            q_v[t, sl] = jnp.where(hi, tok - OFF3,
                                   lax.bitwise_and(tok, NQUAD - 1))
            r_v[t, sl] = lax.shift_left(k, 5)
        return 0

    lax.fori_loop(0, T, _pre, 0, unroll=False)

    gsems = (g0, g1)
    wsems = (w0, w1)

    def _gather(i, par):
        return pltpu.make_async_copy(
            table_hbm.at[q_v.at[i]], buf_v.at[par], gsems[par])

    def _slab_copy(i, par):
        return pltpu.make_async_copy(
            slab_v.at[par], inter_hbm.at[i, pl.ds(n0, NBLK), :], wsems[par])

    def _step(i, par):
        @pl.when(i + 1 < T)
        def _():
            _gather(i + 1, 1 - par).start()

        _gather(i, par).wait()

        @pl.when(i >= 2)
        def _():
            _slab_copy(i, par).wait()

        buf = buf_v.at[par]
        slab = slab_v.at[par]
        for g in range(NBLK // L):
            rsv = r_v[i, pl.ds(g * L, L)]
            for l in range(L):
                rs = rsv[l]
                j = g * L + l
                slab[j, pl.ds(0, L)] = buf[j, pl.ds(rs, L)]
                slab[j, pl.ds(L, L)] = buf[j, pl.ds(rs + L, L)]
        _slab_copy(i, par).start()
        return 0

    def _pair(k, _):
        _step(2 * k, 0)
        _step(2 * k + 1, 1)
        return 0

    _gather(0, 0).start()
    lax.fori_loop(0, T // 2, _pair, 0, unroll=False)

    _slab_copy(T - 2, 0).wait()
    _slab_copy(T - 1, 1).wait()


RPW = 4096
RPG = NQUAD // RPW
_KOFF = (0, NQUAD // RPW, 2 * NQUAD // RPW, OFF3 // RPW)


def _repack_body(x0_ref, x1_ref, x2_ref, x3_ref, out_ref):
    for k, x_ref in enumerate((x0_ref, x1_ref, x2_ref, x3_ref)):
        out_ref[:, k * DIM:(k + 1) * DIM] = x_ref[...].T


_repack = pl.pallas_call(
    _repack_body,
    grid=(RPG,),
    in_specs=[
        pl.BlockSpec((DIM, RPW), lambda j, o=o: (0, o + j)) for o in _KOFF
    ],
    out_specs=pl.BlockSpec((RPW, 128), lambda j: (j, 0)),
    out_shape=jax.ShapeDtypeStruct((NQUAD, 128), jnp.float32),
)


def _proj_body(g_ref, m_ref, w_ref, b_ref, out_ref):
    g = g_ref[0]
    acc = lax.dot_general(
        w_ref[...], g,
        dimension_numbers=(((1,), (1,)), ((), ())),
        preferred_element_type=jnp.float32,
    )
    out_ref[0] = (acc + b_ref[...]) * m_ref[0]


_proj = pl.pallas_call(
    _proj_body,
    grid=(T,),
    in_specs=[
        pl.BlockSpec((1, B, DIM), lambda t: (t, 0, 0)),
        pl.BlockSpec((1, 1, B), lambda t: (t, 0, 0)),
        pl.BlockSpec((OUT_DIM, DIM), lambda t: (0, 0)),
        pl.BlockSpec((OUT_DIM, 1), lambda t: (0, 0)),
    ],
    out_specs=pl.BlockSpec((1, OUT_DIM, B), lambda t: (t, 0, 0)),
    out_shape=jax.ShapeDtypeStruct((T, OUT_DIM, B), jnp.float32),
)


def kernel(tokens, mask, embed_table, W, b):
    tokT = tokens.T.astype(jnp.int32)
    tT = embed_table.T
    table4 = _repack(tT, tT, tT, tT)
    inter = _sc_gather(tokT, table4)
    maskT = mask.T.astype(jnp.float32).reshape(T, 1, B)
    out = _proj(inter, maskT, W, b.reshape(OUT_DIM, 1))
    return jnp.transpose(out, (2, 0, 1)), mask
